# Initial kernel scaffold; baseline (speedup 1.0000x reference)
import functools

import jax
import jax.numpy as jnp
from jax import lax
from jax.experimental import pallas as pl
from jax.experimental.pallas import tpu as pltpu
from jax.experimental.pallas import tpu_sc as plsc

N = 10000
E = 320000
D_NODE = 128
D_EDGE = 16
NODE_FILT = 128
CLASSES = 64

NC = 2
NS = 16
NW = NC * NS
CH = 128
CPT = 79
TPT = CPT * CH
E_PAD = NW * TPT
NPT = N // NS
ZR = 125


def _make_agg(D: int, gather: bool):
    grp = D // 16
    mesh = plsc.VectorSubcoreMesh(core_axis_name="c", subcore_axis_name="s")

    def body(*refs):
        if gather:
            (src_h, dst_h, ew_h, y_h, out_h,
             srcb, dstb, ewb, rows, zbuf, acc) = refs
        else:
            (src_h, dst_h, ew_h, out_h,
             srcb, dstb, ewb, rows, zbuf, acc) = refs
        c = lax.axis_index("c")
        s = lax.axis_index("s")
        w = c * NS + s

        def zrow(r, carry):
            for g in range(grp):
                zbuf[r, pl.ds(g * 16, 16)] = jnp.zeros((16,), jnp.float32)
            return carry
        lax.fori_loop(0, ZR, zrow, 0)

        def zcopy(i, carry):
            pltpu.sync_copy(zbuf, acc.at[pl.ds(s * NPT + i * ZR, ZR)])
            return carry
        lax.fori_loop(0, NPT // ZR, zcopy, 0)
        plsc.subcore_barrier()

        pltpu.sync_copy(src_h.at[w], srcb)
        pltpu.sync_copy(dst_h.at[w], dstb)
        pltpu.sync_copy(ew_h.at[w], ewb)

        def chunk(j, carry):
            if gather:
                pltpu.sync_copy(y_h.at[srcb.at[j]], rows)

                def scale(r, c2):
                    ev = jnp.full((16,), ewb[j, r], jnp.float32)
                    for g in range(grp):
                        rows[r, pl.ds(g * 16, 16)] = (
                            rows[r, pl.ds(g * 16, 16)] * ev)
                    return c2
                lax.fori_loop(0, CH, scale, 0)
            else:
                def fill(r, c2):
                    rows[r, :] = jnp.full((16,), ewb[j, r], jnp.float32)
                    return c2
                lax.fori_loop(0, CH, fill, 0)
            pltpu.sync_copy(rows, acc.at[dstb.at[j]], add=True)
            return carry
        lax.fori_loop(0, CPT, chunk, 0)
        plsc.subcore_barrier()

        pltpu.sync_copy(acc.at[pl.ds(s * NPT, NPT)],
                        out_h.at[c, pl.ds(s * NPT, NPT)])

    return pl.kernel(
        body,
        out_type=jax.ShapeDtypeStruct((NC, N, D), jnp.float32),
        mesh=mesh,
        scratch_types=[
            pltpu.VMEM((CPT, CH), jnp.int32),
            pltpu.VMEM((CPT, CH), jnp.int32),
            pltpu.VMEM((CPT, CH), jnp.float32),
            pltpu.VMEM((CH, D), jnp.float32),
            pltpu.VMEM((ZR, D), jnp.float32),
            pltpu.VMEM_SHARED((N, D), jnp.float32),
        ],
    )


_agg_deg = _make_agg(16, gather=False)
_agg_l1 = _make_agg(D_NODE, gather=True)
_agg_l2 = _make_agg(CLASSES, gather=True)


BE = 4000
BN = 1000


def _ew_body(ex_ref, w1_ref, b1_ref, w2_ref, b2_ref, o_ref):
    h = jnp.maximum(ex_ref[...] @ w1_ref[...] + b1_ref[...], 0.0)
    o_ref[...] = jax.nn.sigmoid(h @ w2_ref[...] + b2_ref[...])


def _l1_body(x_ref, w_ref, degp_ref, y_ref, dis_ref):
    degp = degp_ref[...]
    deg = degp[0, :, 0] + degp[1, :, 0] + 1.0
    dis = lax.rsqrt(deg)[:, None]
    y_ref[...] = dis * (x_ref[...] @ w_ref[...])
    dis_ref[...] = dis


def _l2_body(acc_ref, y_ref, dis_ref, bc1_ref, w_ref, y2_ref):
    a = acc_ref[...]
    dis = dis_ref[...]
    h1 = jnp.maximum(dis * (a[0] + a[1] + y_ref[...]) + bc1_ref[...], 0.0)
    y2_ref[...] = dis * (h1 @ w_ref[...])


def _out_body(acc_ref, y2_ref, dis_ref, bc2_ref, o_ref):
    a = acc_ref[...]
    h2 = dis_ref[...] * (a[0] + a[1] + y2_ref[...]) + bc2_ref[...]
    m = jnp.max(h2, axis=1, keepdims=True)
    lse = m + jnp.log(jnp.sum(jnp.exp(h2 - m), axis=1, keepdims=True))
    o_ref[...] = h2 - lse


def kernel(x, edge_index, edge_x, W1, b1, W2, b2, Wc1, bc1, Wc2, bc2):
    src = edge_index[0]
    dst = edge_index[1]

    ew = pl.pallas_call(
        _ew_body,
        grid=(E // BE,),
        in_specs=[
            pl.BlockSpec((BE, D_EDGE), lambda i: (i, 0)),
            pl.BlockSpec((D_EDGE, D_EDGE), lambda i: (0, 0)),
            pl.BlockSpec((1, D_EDGE), lambda i: (0, 0)),
            pl.BlockSpec((D_EDGE, 1), lambda i: (0, 0)),
            pl.BlockSpec((1, 1), lambda i: (0, 0)),
        ],
        out_specs=pl.BlockSpec((BE, 1), lambda i: (i, 0)),
        out_shape=jax.ShapeDtypeStruct((E, 1), jnp.float32),
    )(edge_x, W1, b1.reshape(1, D_EDGE), W2, b2.reshape(1, 1)).reshape(E)

    pad = E_PAD - E
    srcp = jnp.pad(src, (0, pad)).reshape(NW, CPT, CH)
    dstp = jnp.pad(dst, (0, pad)).reshape(NW, CPT, CH)
    ewp = jnp.pad(ew, (0, pad)).reshape(NW, CPT, CH)

    degp = _agg_deg(srcp, dstp, ewp)

    y, dis = pl.pallas_call(
        _l1_body,
        grid=(N // BN,),
        in_specs=[
            pl.BlockSpec((BN, D_NODE), lambda i: (i, 0)),
            pl.BlockSpec((D_NODE, NODE_FILT), lambda i: (0, 0)),
            pl.BlockSpec((NC, BN, 16), lambda i: (0, i, 0)),
        ],
        out_specs=[
            pl.BlockSpec((BN, NODE_FILT), lambda i: (i, 0)),
            pl.BlockSpec((BN, 1), lambda i: (i, 0)),
        ],
        out_shape=[
            jax.ShapeDtypeStruct((N, NODE_FILT), jnp.float32),
            jax.ShapeDtypeStruct((N, 1), jnp.float32),
        ],
    )(x, Wc1, degp)

    acc1 = _agg_l1(srcp, dstp, ewp, y)

    y2 = pl.pallas_call(
        _l2_body,
        grid=(N // BN,),
        in_specs=[
            pl.BlockSpec((NC, BN, NODE_FILT), lambda i: (0, i, 0)),
            pl.BlockSpec((BN, NODE_FILT), lambda i: (i, 0)),
            pl.BlockSpec((BN, 1), lambda i: (i, 0)),
            pl.BlockSpec((1, NODE_FILT), lambda i: (0, 0)),
            pl.BlockSpec((NODE_FILT, CLASSES), lambda i: (0, 0)),
        ],
        out_specs=pl.BlockSpec((BN, CLASSES), lambda i: (i, 0)),
        out_shape=jax.ShapeDtypeStruct((N, CLASSES), jnp.float32),
    )(acc1, y, dis, bc1.reshape(1, NODE_FILT), Wc2)

    acc2 = _agg_l2(srcp, dstp, ewp, y2)

    out = pl.pallas_call(
        _out_body,
        grid=(N // BN,),
        in_specs=[
            pl.BlockSpec((NC, BN, CLASSES), lambda i: (0, i, 0)),
            pl.BlockSpec((BN, CLASSES), lambda i: (i, 0)),
            pl.BlockSpec((BN, 1), lambda i: (i, 0)),
            pl.BlockSpec((1, CLASSES), lambda i: (0, 0)),
        ],
        out_specs=pl.BlockSpec((BN, CLASSES), lambda i: (i, 0)),
        out_shape=jax.ShapeDtypeStruct((N, CLASSES), jnp.float32),
    )(acc2, y2, dis, bc2.reshape(1, CLASSES))

    return out


# speedup vs baseline: 9.6616x; 9.6616x over previous
import functools

import jax
import jax.numpy as jnp
from jax import lax
from jax.experimental import pallas as pl
from jax.experimental.pallas import tpu as pltpu
from jax.experimental.pallas import tpu_sc as plsc

N = 10000
E = 320000
D_NODE = 128
D_EDGE = 16
NODE_FILT = 128
CLASSES = 64

NC = 2
NS = 16
NW = NC * NS
CH = 128
CPT = 79
TPT = CPT * CH
E_PAD = NW * TPT
N_PAD = 10240
NPT = N_PAD // NS
ZR = 128


def _make_agg(D: int, gather: bool):
    grp = D // 16
    mesh = plsc.VectorSubcoreMesh(core_axis_name="c", subcore_axis_name="s")

    def body(*refs):
        if gather:
            (src_h, dst_h, ew_h, y_h, out_h,
             srcb, dstb, ewb, rows, zbuf, acc) = refs
        else:
            (src_h, dst_h, ew_h, out_h,
             srcb, dstb, ewb, rows, zbuf, acc) = refs
        c = lax.axis_index("c")
        s = lax.axis_index("s")
        w = c * NS + s

        def zrow(r, carry):
            for g in range(grp):
                zbuf[r, pl.ds(g * 16, 16)] = jnp.zeros((16,), jnp.float32)
            return carry
        lax.fori_loop(0, ZR, zrow, 0)

        def zcopy(i, carry):
            pltpu.sync_copy(zbuf, acc.at[pl.ds(s * NPT + i * ZR, ZR)])
            return carry
        lax.fori_loop(0, NPT // ZR, zcopy, 0)
        plsc.subcore_barrier()

        pltpu.sync_copy(src_h.at[w], srcb)
        pltpu.sync_copy(dst_h.at[w], dstb)
        pltpu.sync_copy(ew_h.at[w], ewb)

        def chunk(j, carry):
            if gather:
                pltpu.sync_copy(y_h.at[srcb.at[j]], rows)
            for g in range(CH // 16):
                ev16 = ewb[j, pl.ds(g * 16, 16)]
                for t in range(16):
                    ev = jnp.full((16,), ev16[t], jnp.float32)
                    r = g * 16 + t
                    for q in range(grp):
                        if gather:
                            rows[r, pl.ds(q * 16, 16)] = (
                                rows[r, pl.ds(q * 16, 16)] * ev)
                        else:
                            rows[r, pl.ds(q * 16, 16)] = ev
            pltpu.sync_copy(rows, acc.at[dstb.at[j]], add=True)
            return carry
        lax.fori_loop(0, CPT, chunk, 0)
        plsc.subcore_barrier()

        pltpu.sync_copy(acc.at[pl.ds(s * NPT, NPT)],
                        out_h.at[c, pl.ds(s * NPT, NPT)])

    return pl.kernel(
        body,
        out_type=jax.ShapeDtypeStruct((NC, N_PAD, D), jnp.float32),
        mesh=mesh,
        compiler_params=pltpu.CompilerParams(use_tc_tiling_on_sc=False),
        scratch_types=[
            pltpu.VMEM((CPT, CH), jnp.int32),
            pltpu.VMEM((CPT, CH), jnp.int32),
            pltpu.VMEM((CPT, CH), jnp.float32),
            pltpu.VMEM((CH, D), jnp.float32),
            pltpu.VMEM((ZR, D), jnp.float32),
            pltpu.VMEM_SHARED((N_PAD, D), jnp.float32),
        ],
    )


_agg_deg = _make_agg(16, gather=False)
_agg64 = _make_agg(64, gather=True)


BE = 4000
BN = 1000


def _ew_body(ex_ref, w1_ref, b1_ref, w2_ref, b2_ref, o_ref):
    h = jnp.maximum(ex_ref[...] @ w1_ref[...] + b1_ref[...], 0.0)
    o_ref[...] = jax.nn.sigmoid(h @ w2_ref[...] + b2_ref[...])


def _l1_body(x_ref, w_ref, degp_ref, ylo_ref, yhi_ref, dis_ref):
    degp = degp_ref[...]
    deg = degp[0, :, 0] + degp[1, :, 0] + 1.0
    dis = lax.rsqrt(deg)[:, None]
    y = dis * (x_ref[...] @ w_ref[...])
    ylo_ref[...] = y[:, :64]
    yhi_ref[...] = y[:, 64:]
    dis_ref[...] = dis


def _l2_body(alo_ref, ahi_ref, ylo_ref, yhi_ref, dis_ref, bc1_ref, w_ref,
             y2_ref):
    alo = alo_ref[...]
    ahi = ahi_ref[...]
    dis = dis_ref[...]
    pre = jnp.concatenate([alo[0] + alo[1] + ylo_ref[...],
                           ahi[0] + ahi[1] + yhi_ref[...]], axis=1)
    h1 = jnp.maximum(dis * pre + bc1_ref[...], 0.0)
    y2_ref[...] = dis * (h1 @ w_ref[...])


def _out_body(acc_ref, y2_ref, dis_ref, bc2_ref, o_ref):
    a = acc_ref[...]
    h2 = dis_ref[...] * (a[0] + a[1] + y2_ref[...]) + bc2_ref[...]
    m = jnp.max(h2, axis=1, keepdims=True)
    lse = m + jnp.log(jnp.sum(jnp.exp(h2 - m), axis=1, keepdims=True))
    o_ref[...] = h2 - lse


def kernel(x, edge_index, edge_x, W1, b1, W2, b2, Wc1, bc1, Wc2, bc2):
    src = edge_index[0]
    dst = edge_index[1]

    ew = pl.pallas_call(
        _ew_body,
        grid=(E // BE,),
        in_specs=[
            pl.BlockSpec((BE, D_EDGE), lambda i: (i, 0)),
            pl.BlockSpec((D_EDGE, D_EDGE), lambda i: (0, 0)),
            pl.BlockSpec((1, D_EDGE), lambda i: (0, 0)),
            pl.BlockSpec((D_EDGE, 1), lambda i: (0, 0)),
            pl.BlockSpec((1, 1), lambda i: (0, 0)),
        ],
        out_specs=pl.BlockSpec((BE, 1), lambda i: (i, 0)),
        out_shape=jax.ShapeDtypeStruct((E, 1), jnp.float32),
    )(edge_x, W1, b1.reshape(1, D_EDGE), W2, b2.reshape(1, 1)).reshape(E)

    pad = E_PAD - E
    srcp = jnp.pad(src, (0, pad)).reshape(NW, CPT, CH)
    dstp = jnp.pad(dst, (0, pad)).reshape(NW, CPT, CH)
    ewp = jnp.pad(ew, (0, pad)).reshape(NW, CPT, CH)

    degp = _agg_deg(srcp, dstp, ewp)

    y_lo, y_hi, dis = pl.pallas_call(
        _l1_body,
        grid=(N // BN,),
        in_specs=[
            pl.BlockSpec((BN, D_NODE), lambda i: (i, 0)),
            pl.BlockSpec((D_NODE, NODE_FILT), lambda i: (0, 0)),
            pl.BlockSpec((NC, BN, 16), lambda i: (0, i, 0)),
        ],
        out_specs=[
            pl.BlockSpec((BN, 64), lambda i: (i, 0)),
            pl.BlockSpec((BN, 64), lambda i: (i, 0)),
            pl.BlockSpec((BN, 1), lambda i: (i, 0)),
        ],
        out_shape=[
            jax.ShapeDtypeStruct((N, 64), jnp.float32),
            jax.ShapeDtypeStruct((N, 64), jnp.float32),
            jax.ShapeDtypeStruct((N, 1), jnp.float32),
        ],
    )(x, Wc1, degp)

    acc_lo = _agg64(srcp, dstp, ewp, y_lo)
    acc_hi = _agg64(srcp, dstp, ewp, y_hi)

    y2 = pl.pallas_call(
        _l2_body,
        grid=(N // BN,),
        in_specs=[
            pl.BlockSpec((NC, BN, 64), lambda i: (0, i, 0)),
            pl.BlockSpec((NC, BN, 64), lambda i: (0, i, 0)),
            pl.BlockSpec((BN, 64), lambda i: (i, 0)),
            pl.BlockSpec((BN, 64), lambda i: (i, 0)),
            pl.BlockSpec((BN, 1), lambda i: (i, 0)),
            pl.BlockSpec((1, NODE_FILT), lambda i: (0, 0)),
            pl.BlockSpec((NODE_FILT, CLASSES), lambda i: (0, 0)),
        ],
        out_specs=pl.BlockSpec((BN, CLASSES), lambda i: (i, 0)),
        out_shape=jax.ShapeDtypeStruct((N, CLASSES), jnp.float32),
    )(acc_lo, acc_hi, y_lo, y_hi, dis, bc1.reshape(1, NODE_FILT), Wc2)

    acc2 = _agg64(srcp, dstp, ewp, y2)

    out = pl.pallas_call(
        _out_body,
        grid=(N // BN,),
        in_specs=[
            pl.BlockSpec((NC, BN, CLASSES), lambda i: (0, i, 0)),
            pl.BlockSpec((BN, CLASSES), lambda i: (i, 0)),
            pl.BlockSpec((BN, 1), lambda i: (i, 0)),
            pl.BlockSpec((1, CLASSES), lambda i: (0, 0)),
        ],
        out_specs=pl.BlockSpec((BN, CLASSES), lambda i: (i, 0)),
        out_shape=jax.ShapeDtypeStruct((N, CLASSES), jnp.float32),
    )(acc2, y2, dis, bc2.reshape(1, CLASSES))

    return out
